# baseline (device time: 86003 ns/iter reference)
import jax
import jax.numpy as jnp
from jax import lax
from jax.experimental import pallas as pl
from jax.experimental.pallas import tpu as pltpu

N_DEV = 4
M_CHUNK = 1024
D = 1024


def kernel(partial, gamma):
    def body(x_ref, g_ref, o_ref, send_ref, recv_ref, send_sems, recv_sems):
        p = lax.axis_index("i")
        left = lax.rem(p + N_DEV - 1, N_DEV)
        right = lax.rem(p + 1, N_DEV)

        barrier = pltpu.get_barrier_semaphore()
        for nbr in (left, right):
            pl.semaphore_signal(
                barrier, inc=1,
                device_id=(nbr,), device_id_type=pl.DeviceIdType.MESH,
            )
        pl.semaphore_wait(barrier, 2)

        c0 = lax.rem(p + 3, N_DEV)
        send_ref[0] = x_ref[0, pl.ds(c0 * M_CHUNK, M_CHUNK), :].astype(
            jnp.bfloat16
        )

        for h in range(N_DEV - 1):
            rdma = pltpu.make_async_remote_copy(
                src_ref=send_ref.at[h],
                dst_ref=recv_ref.at[h],
                send_sem=send_sems.at[h],
                recv_sem=recv_sems.at[h],
                device_id=(right,),
                device_id_type=pl.DeviceIdType.MESH,
            )
            rdma.start()
            rdma.wait()

            cr = lax.rem(p + 6 - h, N_DEV)
            local = x_ref[0, pl.ds(cr * M_CHUNK, M_CHUNK), :]
            acc = recv_ref[h].astype(jnp.float32) + local
            if h < N_DEV - 2:
                send_ref[h + 1] = acc.astype(jnp.bfloat16)
            else:
                rms = jnp.sqrt(
                    jnp.mean(acc * acc, axis=-1, keepdims=True) + 1e-6
                )
                o_ref[...] = acc / rms * g_ref[...]

    return pl.pallas_call(
        body,
        out_shape=jax.ShapeDtypeStruct((M_CHUNK, D), jnp.float32),
        in_specs=[
            pl.BlockSpec(memory_space=pltpu.VMEM),
            pl.BlockSpec(memory_space=pltpu.VMEM),
        ],
        out_specs=pl.BlockSpec(memory_space=pltpu.VMEM),
        scratch_shapes=[
            pltpu.VMEM((N_DEV - 1, M_CHUNK, D), jnp.bfloat16),
            pltpu.VMEM((N_DEV - 1, M_CHUNK, D), jnp.bfloat16),
            pltpu.SemaphoreType.DMA((N_DEV - 1,)),
            pltpu.SemaphoreType.DMA((N_DEV - 1,)),
        ],
        compiler_params=pltpu.CompilerParams(collective_id=0),
    )(partial, gamma.reshape(1, D))


# device time: 52513 ns/iter; 1.6377x vs baseline; 1.6377x over previous
import jax
import jax.numpy as jnp
from jax import lax
from jax.experimental import pallas as pl
from jax.experimental.pallas import tpu as pltpu

N_DEV = 4
M_CHUNK = 1024
D = 1024
DH = D // 2


def kernel(partial, gamma):
    def body(
        x_ref, g_ref, o_ref,
        send_cw, recv_cw, send_ccw, recv_ccw,
        ssem_cw, rsem_cw, ssem_ccw, rsem_ccw,
    ):
        p = lax.axis_index("i")
        left = lax.rem(p + N_DEV - 1, N_DEV)
        right = lax.rem(p + 1, N_DEV)

        barrier = pltpu.get_barrier_semaphore()
        for nbr in (left, right):
            pl.semaphore_signal(
                barrier, inc=1,
                device_id=(nbr,), device_id_type=pl.DeviceIdType.MESH,
            )
        pl.semaphore_wait(barrier, 2)

        c0_cw = lax.rem(p + 3, N_DEV)
        c0_ccw = lax.rem(p + 1, N_DEV)
        send_cw[0] = x_ref[0, pl.ds(c0_cw * M_CHUNK, M_CHUNK), :DH].astype(
            jnp.bfloat16
        )
        send_ccw[0] = x_ref[0, pl.ds(c0_ccw * M_CHUNK, M_CHUNK), DH:].astype(
            jnp.bfloat16
        )

        for h in range(N_DEV - 1):
            rdma_cw = pltpu.make_async_remote_copy(
                src_ref=send_cw.at[h],
                dst_ref=recv_cw.at[h],
                send_sem=ssem_cw.at[h],
                recv_sem=rsem_cw.at[h],
                device_id=(right,),
                device_id_type=pl.DeviceIdType.MESH,
            )
            rdma_ccw = pltpu.make_async_remote_copy(
                src_ref=send_ccw.at[h],
                dst_ref=recv_ccw.at[h],
                send_sem=ssem_ccw.at[h],
                recv_sem=rsem_ccw.at[h],
                device_id=(left,),
                device_id_type=pl.DeviceIdType.MESH,
            )
            rdma_cw.start()
            rdma_ccw.start()

            rdma_cw.wait_recv()
            cr_cw = lax.rem(p + 6 - h, N_DEV)
            acc_cw = (
                recv_cw[h].astype(jnp.float32)
                + x_ref[0, pl.ds(cr_cw * M_CHUNK, M_CHUNK), :DH]
            )
            if h < N_DEV - 2:
                send_cw[h + 1] = acc_cw.astype(jnp.bfloat16)

            rdma_ccw.wait_recv()
            cr_ccw = lax.rem(p + 2 + h, N_DEV)
            acc_ccw = (
                recv_ccw[h].astype(jnp.float32)
                + x_ref[0, pl.ds(cr_ccw * M_CHUNK, M_CHUNK), DH:]
            )
            if h < N_DEV - 2:
                send_ccw[h + 1] = acc_ccw.astype(jnp.bfloat16)
            else:
                y = jnp.concatenate([acc_cw, acc_ccw], axis=1)
                rms = jnp.sqrt(
                    jnp.mean(y * y, axis=-1, keepdims=True) + 1e-6
                )
                o_ref[...] = y / rms * g_ref[...]

            rdma_cw.wait_send()
            rdma_ccw.wait_send()

    return pl.pallas_call(
        body,
        out_shape=jax.ShapeDtypeStruct((M_CHUNK, D), jnp.float32),
        in_specs=[
            pl.BlockSpec(memory_space=pltpu.VMEM),
            pl.BlockSpec(memory_space=pltpu.VMEM),
        ],
        out_specs=pl.BlockSpec(memory_space=pltpu.VMEM),
        scratch_shapes=[
            pltpu.VMEM((N_DEV - 1, M_CHUNK, DH), jnp.bfloat16),
            pltpu.VMEM((N_DEV - 1, M_CHUNK, DH), jnp.bfloat16),
            pltpu.VMEM((N_DEV - 1, M_CHUNK, DH), jnp.bfloat16),
            pltpu.VMEM((N_DEV - 1, M_CHUNK, DH), jnp.bfloat16),
            pltpu.SemaphoreType.DMA((N_DEV - 1,)),
            pltpu.SemaphoreType.DMA((N_DEV - 1,)),
            pltpu.SemaphoreType.DMA((N_DEV - 1,)),
            pltpu.SemaphoreType.DMA((N_DEV - 1,)),
        ],
        compiler_params=pltpu.CompilerParams(collective_id=0),
    )(partial, gamma.reshape(1, D))


# device time: 46604 ns/iter; 1.8454x vs baseline; 1.1268x over previous
import jax
import jax.numpy as jnp
from jax import lax
from jax.experimental import pallas as pl
from jax.experimental.pallas import tpu as pltpu

N_DEV = 4
M_CHUNK = 1024
D = 1024
DH = D // 2
SUB = 4
R = M_CHUNK // SUB


def kernel(partial, gamma):
    def body(
        x_ref, g_ref, o_ref,
        send_cw, recv_cw, send_ccw, recv_ccw,
        ssem_cw, rsem_cw, ssem_ccw, rsem_ccw,
    ):
        p = lax.axis_index("i")
        left = lax.rem(p + N_DEV - 1, N_DEV)
        right = lax.rem(p + 1, N_DEV)

        def rdma(h, j, cw):
            if cw:
                return pltpu.make_async_remote_copy(
                    src_ref=send_cw.at[h, j],
                    dst_ref=recv_cw.at[h, j],
                    send_sem=ssem_cw.at[h, j],
                    recv_sem=rsem_cw.at[h, j],
                    device_id=(right,),
                    device_id_type=pl.DeviceIdType.MESH,
                )
            return pltpu.make_async_remote_copy(
                src_ref=send_ccw.at[h, j],
                dst_ref=recv_ccw.at[h, j],
                send_sem=ssem_ccw.at[h, j],
                recv_sem=rsem_ccw.at[h, j],
                device_id=(left,),
                device_id_type=pl.DeviceIdType.MESH,
            )

        barrier = pltpu.get_barrier_semaphore()
        for nbr in (left, right):
            pl.semaphore_signal(
                barrier, inc=1,
                device_id=(nbr,), device_id_type=pl.DeviceIdType.MESH,
            )
        pl.semaphore_wait(barrier, 2)

        c0_cw = lax.rem(p + 3, N_DEV)
        c0_ccw = lax.rem(p + 1, N_DEV)
        for j in range(SUB):
            send_cw[0, j] = x_ref[
                0, pl.ds(c0_cw * M_CHUNK + j * R, R), :DH
            ].astype(jnp.bfloat16)
            rdma(0, j, True).start()
            send_ccw[0, j] = x_ref[
                0, pl.ds(c0_ccw * M_CHUNK + j * R, R), DH:
            ].astype(jnp.bfloat16)
            rdma(0, j, False).start()

        for h in range(N_DEV - 1):
            cr_cw = lax.rem(p + 6 - h, N_DEV)
            cr_ccw = lax.rem(p + 2 + h, N_DEV)
            for j in range(SUB):
                rdma(h, j, True).wait_recv()
                acc_cw = (
                    recv_cw[h, j].astype(jnp.float32)
                    + x_ref[0, pl.ds(cr_cw * M_CHUNK + j * R, R), :DH]
                )
                if h < N_DEV - 2:
                    send_cw[h + 1, j] = acc_cw.astype(jnp.bfloat16)
                    rdma(h + 1, j, True).start()

                rdma(h, j, False).wait_recv()
                acc_ccw = (
                    recv_ccw[h, j].astype(jnp.float32)
                    + x_ref[0, pl.ds(cr_ccw * M_CHUNK + j * R, R), DH:]
                )
                if h < N_DEV - 2:
                    send_ccw[h + 1, j] = acc_ccw.astype(jnp.bfloat16)
                    rdma(h + 1, j, False).start()
                else:
                    y = jnp.concatenate([acc_cw, acc_ccw], axis=1)
                    rms = jnp.sqrt(
                        jnp.mean(y * y, axis=-1, keepdims=True) + 1e-6
                    )
                    o_ref[j * R:(j + 1) * R, :] = y / rms * g_ref[...]

        for h in range(N_DEV - 1):
            for j in range(SUB):
                rdma(h, j, True).wait_send()
                rdma(h, j, False).wait_send()

    return pl.pallas_call(
        body,
        out_shape=jax.ShapeDtypeStruct((M_CHUNK, D), jnp.float32),
        in_specs=[
            pl.BlockSpec(memory_space=pltpu.VMEM),
            pl.BlockSpec(memory_space=pltpu.VMEM),
        ],
        out_specs=pl.BlockSpec(memory_space=pltpu.VMEM),
        scratch_shapes=[
            pltpu.VMEM((N_DEV - 1, SUB, R, DH), jnp.bfloat16),
            pltpu.VMEM((N_DEV - 1, SUB, R, DH), jnp.bfloat16),
            pltpu.VMEM((N_DEV - 1, SUB, R, DH), jnp.bfloat16),
            pltpu.VMEM((N_DEV - 1, SUB, R, DH), jnp.bfloat16),
            pltpu.SemaphoreType.DMA((N_DEV - 1, SUB)),
            pltpu.SemaphoreType.DMA((N_DEV - 1, SUB)),
            pltpu.SemaphoreType.DMA((N_DEV - 1, SUB)),
            pltpu.SemaphoreType.DMA((N_DEV - 1, SUB)),
        ],
        compiler_params=pltpu.CompilerParams(collective_id=0),
    )(partial, gamma.reshape(1, D))
